# baseline (speedup 1.0000x reference)
import functools

import jax
import jax.numpy as jnp
from jax import lax
from jax.experimental import pallas as pl
from jax.experimental.pallas import tpu as pltpu
from jax.experimental.pallas import tpu_sc as plsc

N_EMBED = 8192
EMBED_DIM = 256
BETA = 0.25
N_TOKENS = 16384

TM = 128

SC_CORES = 2
SC_SUBCORES = 16
SC_WORKERS = SC_CORES * SC_SUBCORES
GATHER_CHUNK = 128


CN = 1024


def _w2_body(w_ref, w2_ref):
    w = w_ref[...]
    w2_ref[...] = jnp.sum(w * w, axis=1)[None, :]


def _w2(weight):
    return pl.pallas_call(
        _w2_body,
        out_shape=jax.ShapeDtypeStruct((1, N_EMBED), jnp.float32),
    )(weight)


def _dist_argmin_body(z_ref, w_ref, w2_ref, idx_ref, dmin_ref):
    zt = z_ref[...]
    nrm = jnp.sqrt(jnp.sum(zt * zt, axis=1, keepdims=True))
    zn = zt / jnp.clip(nrm, 1e-12, None)
    z2 = jnp.sum(zn * zn, axis=1, keepdims=True)
    zm = zn * (-2.0)
    best_d = None
    best_i = None
    for c in range(0, N_EMBED, CN):
        dotn = lax.dot_general(
            zm, w_ref[c:c + CN, :], (((1,), (1,)), ((), ())),
            precision=lax.Precision.DEFAULT,
            preferred_element_type=jnp.float32,
        )
        d = (z2 + w2_ref[:, c:c + CN]) + dotn
        dmin_c = jnp.min(d, axis=1)
        ids = lax.broadcasted_iota(jnp.int32, d.shape, 1)
        cand = jnp.where(d == dmin_c[:, None], ids, jnp.int32(CN))
        idx_c = jnp.min(cand, axis=1) + c
        if best_d is None:
            best_d, best_i = dmin_c, idx_c
        else:
            best_i = jnp.where(dmin_c < best_d, idx_c, best_i)
            best_d = jnp.minimum(best_d, dmin_c)
    idx_ref[...] = best_i
    dmin_ref[...] = best_d


def _dist_argmin(z, weight):
    w2 = _w2(weight)
    grid = (N_TOKENS // TM,)
    return pl.pallas_call(
        _dist_argmin_body,
        grid=grid,
        in_specs=[
            pl.BlockSpec((TM, EMBED_DIM), lambda i: (i, 0)),
            pl.BlockSpec((N_EMBED, EMBED_DIM), lambda i: (0, 0)),
            pl.BlockSpec((1, N_EMBED), lambda i: (0, 0)),
        ],
        out_specs=[
            pl.BlockSpec((TM,), lambda i: (i,)),
            pl.BlockSpec((TM,), lambda i: (i,)),
        ],
        out_shape=[
            jax.ShapeDtypeStruct((N_TOKENS,), jnp.int32),
            jax.ShapeDtypeStruct((N_TOKENS,), jnp.float32),
        ],
        compiler_params=pltpu.CompilerParams(
            dimension_semantics=("parallel",),
        ),
    )(z, weight, w2)


def _sc_gather(weight, idx):
    b_per_w = N_TOKENS // SC_WORKERS
    mesh = plsc.VectorSubcoreMesh(core_axis_name="c", subcore_axis_name="s")

    @functools.partial(
        pl.kernel,
        mesh=mesh,
        out_type=jax.ShapeDtypeStruct((N_TOKENS, EMBED_DIM), jnp.float32),
        scratch_types=[
            pltpu.VMEM((GATHER_CHUNK,), jnp.int32),
            pltpu.VMEM((GATHER_CHUNK, EMBED_DIM), jnp.float32),
            pltpu.SemaphoreType.DMA,
        ],
    )
    def gather_kernel(table_hbm, idx_hbm, out_hbm, idx_v, rows_v, sem):
        wid = lax.axis_index("s") * SC_CORES + lax.axis_index("c")
        base = wid * b_per_w

        @pl.loop(0, b_per_w, step=GATHER_CHUNK)
        def _(c):
            pltpu.sync_copy(idx_hbm.at[pl.ds(base + c, GATHER_CHUNK)], idx_v)
            pltpu.async_copy(table_hbm.at[idx_v], rows_v, sem).wait()
            pltpu.sync_copy(rows_v, out_hbm.at[pl.ds(base + c, GATHER_CHUNK)])

    return gather_kernel(weight, idx)


def kernel(z, weight):
    idx, dmin = _dist_argmin(z, weight)
    z_q = _sc_gather(weight, idx)
    loss = (BETA / EMBED_DIM) * jnp.mean(dmin)
    return (z_q, loss, idx)


# speedup vs baseline: 1.4824x; 1.4824x over previous
import functools

import jax
import jax.numpy as jnp
from jax import lax
from jax.experimental import pallas as pl
from jax.experimental.pallas import tpu as pltpu
from jax.experimental.pallas import tpu_sc as plsc

N_EMBED = 8192
EMBED_DIM = 256
BETA = 0.25
N_TOKENS = 16384

TM = 256

SC_CORES = 2
SC_SUBCORES = 16
SC_WORKERS = SC_CORES * SC_SUBCORES
GATHER_CHUNK = 128


CN = 1024


def _w2_body(w_ref, w2_ref):
    w = w_ref[...]
    w2_ref[...] = jnp.sum(w * w, axis=1, keepdims=True) + jnp.zeros(
        (1, TM), jnp.float32)


def _w2(weight):
    return pl.pallas_call(
        _w2_body,
        out_shape=jax.ShapeDtypeStruct((N_EMBED, TM), jnp.float32),
    )(weight)


def _dist_argmin_body(z_ref, w_ref, w2_ref, idx_ref, dmin_ref):
    zt = z_ref[...]
    nrm = jnp.sqrt(jnp.sum(zt * zt, axis=1, keepdims=True))
    zn = zt / jnp.clip(nrm, 1e-12, None)
    z2 = jnp.sum(zn * zn, axis=1, keepdims=True)
    z2r = z2.reshape(1, TM)
    zm = zn * (-2.0)
    best_d = None
    best_i = None
    for c in range(0, N_EMBED, CN):
        dotn = lax.dot_general(
            w_ref[c:c + CN, :], zm, (((1,), (1,)), ((), ())),
            precision=lax.Precision.DEFAULT,
            preferred_element_type=jnp.float32,
        )
        d = (w2_ref[c:c + CN, :] + z2r) + dotn
        dmin_c = jnp.min(d, axis=0)
        ids = lax.broadcasted_iota(jnp.int32, d.shape, 0)
        cand = jnp.where(d == dmin_c[None, :], ids, jnp.int32(CN))
        idx_c = jnp.min(cand, axis=0) + c
        if best_d is None:
            best_d, best_i = dmin_c, idx_c
        else:
            best_i = jnp.where(dmin_c < best_d, idx_c, best_i)
            best_d = jnp.minimum(best_d, dmin_c)
    idx_ref[...] = best_i
    dmin_ref[...] = best_d


def _dist_argmin(z, weight):
    w2 = _w2(weight)
    grid = (N_TOKENS // TM,)
    return pl.pallas_call(
        _dist_argmin_body,
        grid=grid,
        in_specs=[
            pl.BlockSpec((TM, EMBED_DIM), lambda i: (i, 0)),
            pl.BlockSpec((N_EMBED, EMBED_DIM), lambda i: (0, 0)),
            pl.BlockSpec((N_EMBED, TM), lambda i: (0, 0)),
        ],
        out_specs=[
            pl.BlockSpec((TM,), lambda i: (i,)),
            pl.BlockSpec((TM,), lambda i: (i,)),
        ],
        out_shape=[
            jax.ShapeDtypeStruct((N_TOKENS,), jnp.int32),
            jax.ShapeDtypeStruct((N_TOKENS,), jnp.float32),
        ],
        compiler_params=pltpu.CompilerParams(
            dimension_semantics=("parallel",),
        ),
    )(z, weight, w2)


def _sc_gather(weight, idx):
    b_per_w = N_TOKENS // SC_WORKERS
    mesh = plsc.VectorSubcoreMesh(core_axis_name="c", subcore_axis_name="s")

    @functools.partial(
        pl.kernel,
        mesh=mesh,
        out_type=jax.ShapeDtypeStruct((N_TOKENS, EMBED_DIM), jnp.float32),
        scratch_types=[
            pltpu.VMEM((GATHER_CHUNK,), jnp.int32),
            pltpu.VMEM((GATHER_CHUNK, EMBED_DIM), jnp.float32),
            pltpu.SemaphoreType.DMA,
        ],
    )
    def gather_kernel(table_hbm, idx_hbm, out_hbm, idx_v, rows_v, sem):
        wid = lax.axis_index("s") * SC_CORES + lax.axis_index("c")
        base = wid * b_per_w

        @pl.loop(0, b_per_w, step=GATHER_CHUNK)
        def _(c):
            pltpu.sync_copy(idx_hbm.at[pl.ds(base + c, GATHER_CHUNK)], idx_v)
            pltpu.async_copy(table_hbm.at[idx_v], rows_v, sem).wait()
            pltpu.sync_copy(rows_v, out_hbm.at[pl.ds(base + c, GATHER_CHUNK)])

    return gather_kernel(weight, idx)


def kernel(z, weight):
    idx, dmin = _dist_argmin(z, weight)
    z_q = _sc_gather(weight, idx)
    loss = (BETA / EMBED_DIM) * jnp.mean(dmin)
    return (z_q, loss, idx)
